# Initial kernel scaffold; baseline (speedup 1.0000x reference)
import jax
import jax.numpy as jnp
from jax.experimental import pallas as pl


def kernel(points, costs, default_cost):
    raise NotImplementedError("write your pallas kernel here")


# speedup vs baseline: 1.4254x; 1.4254x over previous
import functools

import jax
import jax.numpy as jnp
from jax import lax
from jax.experimental import pallas as pl
from jax.experimental.pallas import tpu as pltpu
from jax.experimental.pallas import tpu_sc as plsc

H = 512
W = 512
G = H * W
NC = 2
NS = 16
NW = NC * NS
L = 16

CH = 2000
SLABS = ((0, 88064), (88064, 88064), (176128, 86016))
SMAX = 88064


def _fill(ref, n, vec):
    def body(i, _):
        ref[pl.ds(i * L, L)] = vec
        return 0
    lax.fori_loop(0, n // L, body, 0)


def _rmw_min(grid_ref, idx_vec, cost_vec, slab_lo, slab_n):
    m = (idx_vec >= slab_lo) & (idx_vec < slab_lo + slab_n)
    li = jnp.clip(idx_vec - slab_lo, 0, slab_n - 1)
    cur = plsc.load_gather(grid_ref, [li], mask=m)
    val = jnp.minimum(cost_vec, jnp.where(m, cur, cost_vec))
    plsc.store_scatter(grid_ref, [li], val, mask=m)
    chk = plsc.load_gather(grid_ref, [li], mask=m)
    need = m & (chk > val)

    def wbody(nd):
        plsc.store_scatter(grid_ref, [li], val, mask=nd)
        c2 = plsc.load_gather(grid_ref, [li], mask=nd)
        return nd & (c2 > val)

    lax.while_loop(lambda nd: jnp.any(nd), wbody, need)


@functools.partial(jax.jit, static_argnames=("n_points",))
def _sc_scatter(points_flat, costs, *, n_points):
    mesh = plsc.VectorSubcoreMesh(
        core_axis_name="c", subcore_axis_name="s", num_cores=NC, num_subcores=NS
    )
    pw = n_points // NW
    n_chunks = pw // CH

    def body(pts_hbm, costs_hbm, grids_hbm, counts_hbm, idx_hbm,
             grid_v, pts_v, cost_v, idx_v, ones_v, zero_v, counts_sp):
        cid = lax.axis_index("c")
        sid = lax.axis_index("s")
        wid = sid * NC + cid
        base = wid * pw
        ev = 2 * lax.iota(jnp.int32, L)
        inf16 = jnp.full((L,), jnp.inf, jnp.float32)

        _fill(ones_v, CH, jnp.ones((L,), jnp.int32))
        _fill(zero_v, 2048, jnp.zeros((L,), jnp.int32))

        stripe = G // NS
        def zloop(k, _):
            pltpu.sync_copy(zero_v,
                            counts_sp.at[pl.ds(sid * stripe + k * 2048, 2048)])
            return 0
        lax.fori_loop(0, stripe // 2048, zloop, 0)
        plsc.subcore_barrier()

        for p, (slab_lo, slab_n) in enumerate(SLABS):
            _fill(grid_v, SMAX, inf16)

            def chunk0(c, _):
                start = pl.multiple_of(base + c * CH, 8)
                pltpu.sync_copy(pts_hbm.at[pl.ds(pl.multiple_of(2 * start, 8),
                                                 2 * CH)], pts_v)
                pltpu.sync_copy(costs_hbm.at[pl.ds(start, CH)], cost_v)

                def vloop(j, _):
                    b2 = j * 2 * L
                    xs = plsc.load_gather(pts_v, [b2 + ev])
                    ys = plsc.load_gather(pts_v, [b2 + ev + 1])
                    ix = jnp.clip((xs + 0.5).astype(jnp.int32), 0, W - 1)
                    iy = jnp.clip((ys + 0.5).astype(jnp.int32), 0, H - 1)
                    iv = iy * W + ix
                    idx_v[pl.ds(j * L, L)] = iv
                    cv = cost_v[pl.ds(j * L, L)]
                    _rmw_min(grid_v, iv, cv, slab_lo, slab_n)
                    return 0
                lax.fori_loop(0, CH // L, vloop, 0)

                pltpu.sync_copy(idx_v, idx_hbm.at[pl.ds(start, CH)])
                pltpu.sync_copy(ones_v, counts_sp.at[idx_v], add=True)
                return 0

            def chunkp(c, _):
                start = pl.multiple_of(base + c * CH, 8)
                pltpu.sync_copy(idx_hbm.at[pl.ds(start, CH)], idx_v)
                pltpu.sync_copy(costs_hbm.at[pl.ds(start, CH)], cost_v)

                def vloop(j, _):
                    iv = idx_v[pl.ds(j * L, L)]
                    cv = cost_v[pl.ds(j * L, L)]
                    _rmw_min(grid_v, iv, cv, slab_lo, slab_n)
                    return 0
                lax.fori_loop(0, CH // L, vloop, 0)
                return 0

            lax.fori_loop(0, n_chunks, chunk0 if p == 0 else chunkp, 0)

            pltpu.sync_copy(grid_v.at[pl.ds(0, slab_n)],
                            grids_hbm.at[wid, pl.ds(slab_lo, slab_n)])

            if p == 0:
                plsc.subcore_barrier()
                pltpu.sync_copy(counts_sp.at[pl.ds(sid * stripe, stripe)],
                                counts_hbm.at[cid, pl.ds(sid * stripe, stripe)])

    f = pl.kernel(
        body,
        out_type=(
            jax.ShapeDtypeStruct((NW, G), jnp.float32),
            jax.ShapeDtypeStruct((NC, G), jnp.int32),
            jax.ShapeDtypeStruct((n_points,), jnp.int32),
        ),
        mesh=mesh,
        compiler_params=pltpu.CompilerParams(needs_layout_passes=False),
        scratch_types=(
            pltpu.VMEM((SMAX,), jnp.float32),
            pltpu.VMEM((2 * CH,), jnp.float32),
            pltpu.VMEM((CH,), jnp.float32),
            pltpu.VMEM((CH,), jnp.int32),
            pltpu.VMEM((CH,), jnp.int32),
            pltpu.VMEM((2048,), jnp.int32),
            pltpu.VMEM_SHARED((G,), jnp.int32),
        ),
    )
    return f(points_flat, costs)


def _merge_body(dflt_ref, grids_ref, counts_ref, cost_ref, mask_ref):
    g = grids_ref[...]
    mn = jnp.min(g, axis=0)
    c = counts_ref[0] + counts_ref[1]
    mask_ref[...] = c - 1
    cost_ref[...] = jnp.where(c >= 1, mn, dflt_ref[0, 0])


_R = 64

_merge = pl.pallas_call(
    _merge_body,
    grid=(H // _R,),
    in_specs=[
        pl.BlockSpec(memory_space=pltpu.SMEM),
        pl.BlockSpec((NW, _R, W), lambda i: (0, i, 0)),
        pl.BlockSpec((NC, _R, W), lambda i: (0, i, 0)),
    ],
    out_specs=[
        pl.BlockSpec((_R, W), lambda i: (i, 0)),
        pl.BlockSpec((_R, W), lambda i: (i, 0)),
    ],
    out_shape=[
        jax.ShapeDtypeStruct((H, W), jnp.float32),
        jax.ShapeDtypeStruct((H, W), jnp.int32),
    ],
)


def kernel(points, costs, default_cost):
    n = costs.shape[0]
    grids, counts, _ = _sc_scatter(points.reshape(-1), costs, n_points=n)
    cost, mask = _merge(
        default_cost.reshape(1, 1),
        grids.reshape(NW, H, W),
        counts.reshape(NC, H, W),
    )
    return cost, mask


# speedup vs baseline: 6.6739x; 4.6820x over previous
import functools

import jax
import jax.numpy as jnp
from jax import lax
from jax.experimental import pallas as pl
from jax.experimental.pallas import tpu as pltpu
from jax.experimental.pallas import tpu_sc as plsc

H = 512
W = 512
G = H * W
NC = 2
NS = 16
NW = NC * NS
L = 16

CH = 2000
SLABS = ((0, 88064), (88064, 88064), (176128, 86016))
SMAX = 88064


def _fill(ref, n, vec):
    def body(i, _):
        ref[pl.ds(i * L, L)] = vec
        return 0
    lax.fori_loop(0, n // L, body, 0)


def _rmw_min(grid_ref, idx_vec, cost_vec, slab_lo, slab_n):
    m = (idx_vec >= slab_lo) & (idx_vec < slab_lo + slab_n)
    li = jnp.clip(idx_vec - slab_lo, 0, slab_n - 1)
    cur = plsc.load_gather(grid_ref, [li], mask=m)
    val = jnp.minimum(cost_vec, jnp.where(m, cur, cost_vec))
    plsc.store_scatter(grid_ref, [li], val, mask=m)
    chk = plsc.load_gather(grid_ref, [li], mask=m)
    need = m & (chk > val)

    def wbody(nd):
        plsc.store_scatter(grid_ref, [li], val, mask=nd)
        c2 = plsc.load_gather(grid_ref, [li], mask=nd)
        return nd & (c2 > val)

    lax.while_loop(lambda nd: jnp.any(nd), wbody, need)


@functools.partial(jax.jit, static_argnames=("n_points",))
def _sc_scatter(xs, ys, costs, *, n_points):
    mesh = plsc.VectorSubcoreMesh(
        core_axis_name="c", subcore_axis_name="s", num_cores=NC, num_subcores=NS
    )
    pw = n_points // NW
    n_chunks = pw // CH

    def body(xs_hbm, ys_hbm, costs_hbm, grids_hbm, counts_hbm, idx_hbm,
             grid_v, xs_v, ys_v, cost_v, idx_v, ones_v, zero_v, counts_sp):
        cid = lax.axis_index("c")
        sid = lax.axis_index("s")
        wid = sid * NC + cid
        base = wid * pw
        inf16 = jnp.full((L,), jnp.inf, jnp.float32)

        _fill(ones_v, CH, jnp.ones((L,), jnp.int32))
        _fill(zero_v, 2048, jnp.zeros((L,), jnp.int32))

        stripe = G // NS
        def zloop(k, _):
            pltpu.sync_copy(zero_v,
                            counts_sp.at[pl.ds(sid * stripe + k * 2048, 2048)])
            return 0
        lax.fori_loop(0, stripe // 2048, zloop, 0)
        plsc.subcore_barrier()

        for p, (slab_lo, slab_n) in enumerate(SLABS):
            _fill(grid_v, SMAX, inf16)

            def chunk0(c, _):
                start = pl.multiple_of(base + c * CH, 8)
                pltpu.sync_copy(xs_hbm.at[pl.ds(start, CH)], xs_v)
                pltpu.sync_copy(ys_hbm.at[pl.ds(start, CH)], ys_v)
                pltpu.sync_copy(costs_hbm.at[pl.ds(start, CH)], cost_v)

                def vloop(j, _):
                    xv = xs_v[pl.ds(j * L, L)]
                    yv = ys_v[pl.ds(j * L, L)]
                    ix = jnp.clip((xv + 0.5).astype(jnp.int32), 0, W - 1)
                    iy = jnp.clip((yv + 0.5).astype(jnp.int32), 0, H - 1)
                    iv = iy * W + ix
                    idx_v[pl.ds(j * L, L)] = iv
                    cv = cost_v[pl.ds(j * L, L)]
                    _rmw_min(grid_v, iv, cv, slab_lo, slab_n)
                    return 0
                lax.fori_loop(0, CH // L, vloop, 0)

                pltpu.sync_copy(idx_v, idx_hbm.at[pl.ds(start, CH)])
                pltpu.sync_copy(ones_v, counts_sp.at[idx_v], add=True)
                return 0

            def chunkp(c, _):
                start = pl.multiple_of(base + c * CH, 8)
                pltpu.sync_copy(idx_hbm.at[pl.ds(start, CH)], idx_v)
                pltpu.sync_copy(costs_hbm.at[pl.ds(start, CH)], cost_v)

                def vloop(j, _):
                    iv = idx_v[pl.ds(j * L, L)]
                    cv = cost_v[pl.ds(j * L, L)]
                    _rmw_min(grid_v, iv, cv, slab_lo, slab_n)
                    return 0
                lax.fori_loop(0, CH // L, vloop, 0)
                return 0

            lax.fori_loop(0, n_chunks, chunk0 if p == 0 else chunkp, 0)

            pltpu.sync_copy(grid_v.at[pl.ds(0, slab_n)],
                            grids_hbm.at[wid, pl.ds(slab_lo, slab_n)])

            if p == 0:
                plsc.subcore_barrier()
                pltpu.sync_copy(counts_sp.at[pl.ds(sid * stripe, stripe)],
                                counts_hbm.at[cid, pl.ds(sid * stripe, stripe)])

    f = pl.kernel(
        body,
        out_type=(
            jax.ShapeDtypeStruct((NW, G), jnp.float32),
            jax.ShapeDtypeStruct((NC, G), jnp.int32),
            jax.ShapeDtypeStruct((n_points,), jnp.int32),
        ),
        mesh=mesh,
        compiler_params=pltpu.CompilerParams(needs_layout_passes=False),
        scratch_types=(
            pltpu.VMEM((SMAX,), jnp.float32),
            pltpu.VMEM((CH,), jnp.float32),
            pltpu.VMEM((CH,), jnp.float32),
            pltpu.VMEM((CH,), jnp.float32),
            pltpu.VMEM((CH,), jnp.int32),
            pltpu.VMEM((CH,), jnp.int32),
            pltpu.VMEM((2048,), jnp.int32),
            pltpu.VMEM_SHARED((G,), jnp.int32),
        ),
    )
    return f(xs, ys, costs)


def _merge_body(dflt_ref, grids_ref, counts_ref, cost_ref, mask_ref):
    g = grids_ref[...]
    mn = jnp.min(g, axis=0)
    c = counts_ref[0] + counts_ref[1]
    mask_ref[...] = c - 1
    cost_ref[...] = jnp.where(c >= 1, mn, dflt_ref[0, 0])


_R = 64

_merge = pl.pallas_call(
    _merge_body,
    grid=(H // _R,),
    in_specs=[
        pl.BlockSpec(memory_space=pltpu.SMEM),
        pl.BlockSpec((NW, _R, W), lambda i: (0, i, 0)),
        pl.BlockSpec((NC, _R, W), lambda i: (0, i, 0)),
    ],
    out_specs=[
        pl.BlockSpec((_R, W), lambda i: (i, 0)),
        pl.BlockSpec((_R, W), lambda i: (i, 0)),
    ],
    out_shape=[
        jax.ShapeDtypeStruct((H, W), jnp.float32),
        jax.ShapeDtypeStruct((H, W), jnp.int32),
    ],
)


def kernel(points, costs, default_cost):
    n = costs.shape[0]
    grids, counts, _ = _sc_scatter(points[:, 0], points[:, 1], costs, n_points=n)
    cost, mask = _merge(
        default_cost.reshape(1, 1),
        grids.reshape(NW, H, W),
        counts.reshape(NC, H, W),
    )
    return cost, mask


# speedup vs baseline: 8.6796x; 1.3005x over previous
import functools

import jax
import jax.numpy as jnp
from jax import lax
from jax.experimental import pallas as pl
from jax.experimental.pallas import tpu as pltpu
from jax.experimental.pallas import tpu_sc as plsc

H = 512
W = 512
G = H * W
NC = 2
NS = 16
NW = NC * NS
L = 16

CH = 2000
SLABS = ((0, 88064), (88064, 88064), (176128, 86016))
SMAX = 88064


def _fill(ref, n, vec):
    nv = n // L
    unroll = 8 if nv % 8 == 0 else (5 if nv % 5 == 0 else 1)
    step = L * unroll
    assert n % step == 0

    def body(i, _):
        for t in range(unroll):
            ref[pl.ds(i * step + t * L, L)] = vec
        return 0

    lax.fori_loop(0, n // step, body, 0)


def _rmw_min(grid_ref, idx_vec, cost_vec, slab_lo, slab_n):
    m = (idx_vec >= slab_lo) & (idx_vec < slab_lo + slab_n)
    li = jnp.clip(idx_vec - slab_lo, 0, slab_n - 1)
    cur = plsc.load_gather(grid_ref, [li], mask=m)
    val = jnp.minimum(cost_vec, jnp.where(m, cur, cost_vec))
    plsc.store_scatter(grid_ref, [li], val, mask=m)
    chk = plsc.load_gather(grid_ref, [li], mask=m)
    need = m & (chk > val)

    def wbody(nd):
        plsc.store_scatter(grid_ref, [li], val, mask=nd)
        c2 = plsc.load_gather(grid_ref, [li], mask=nd)
        return nd & (c2 > val)

    lax.while_loop(lambda nd: jnp.any(nd), wbody, need)


@functools.partial(jax.jit, static_argnames=("n_points",))
def _sc_scatter(xs, ys, costs, *, n_points):
    mesh = plsc.VectorSubcoreMesh(
        core_axis_name="c", subcore_axis_name="s", num_cores=NC, num_subcores=NS
    )
    pw = n_points // NW
    n_chunks = pw // CH
    assert n_chunks % 2 == 0

    def body(xs_hbm, ys_hbm, costs_hbm, grids_hbm, counts_hbm, idx_hbm,
             grid_v, bufs_f, bufs_i, ones_v, sem, counts_sp):
        cid = lax.axis_index("c")
        sid = lax.axis_index("s")
        wid = sid * NC + cid
        base = wid * pw
        inf16 = jnp.full((L,), jnp.inf, jnp.float32)

        _fill(ones_v, CH, jnp.ones((L,), jnp.int32))
        _fill(bufs_i[0], CH, jnp.zeros((L,), jnp.int32))
        stripe = G // NS
        def zloop(k, _):
            pltpu.sync_copy(bufs_i[0].at[pl.ds(0, 1024)],
                            counts_sp.at[pl.ds(sid * stripe + k * 1024, 1024)])
            return 0
        lax.fori_loop(0, stripe // 1024, zloop, 0)
        plsc.subcore_barrier()

        def chunk_start(p, c, k):
            start = pl.multiple_of(base + c * CH, 8)
            if p == 0:
                pltpu.async_copy(xs_hbm.at[pl.ds(start, CH)], bufs_f[3 * k], sem)
                pltpu.async_copy(ys_hbm.at[pl.ds(start, CH)], bufs_f[3 * k + 1], sem)
            else:
                pltpu.async_copy(idx_hbm.at[pl.ds(start, CH)], bufs_i[k], sem)
            pltpu.async_copy(costs_hbm.at[pl.ds(start, CH)], bufs_f[3 * k + 2], sem)

        def chunk_wait(p, k):
            if p == 0:
                pltpu.make_async_copy(xs_hbm.at[pl.ds(0, CH)], bufs_f[3 * k], sem).wait()
                pltpu.make_async_copy(xs_hbm.at[pl.ds(0, CH)], bufs_f[3 * k + 1], sem).wait()
            else:
                pltpu.make_async_copy(idx_hbm.at[pl.ds(0, CH)], bufs_i[k], sem).wait()
            pltpu.make_async_copy(xs_hbm.at[pl.ds(0, CH)], bufs_f[3 * k + 2], sem).wait()

        for p, (slab_lo, slab_n) in enumerate(SLABS):
            _fill(grid_v, SMAX, inf16)

            cost_b = (bufs_f[2], bufs_f[5])
            if p == 0:
                def process(c, k):
                    xv_b, yv_b = bufs_f[3 * k], bufs_f[3 * k + 1]
                    cv_b, iv_b = cost_b[k], bufs_i[k]

                    def vloop(j, _):
                        xv = xv_b[pl.ds(j * L, L)]
                        yv = yv_b[pl.ds(j * L, L)]
                        ix = jnp.clip((xv + 0.5).astype(jnp.int32), 0, W - 1)
                        iy = jnp.clip((yv + 0.5).astype(jnp.int32), 0, H - 1)
                        iv = iy * W + ix
                        iv_b[pl.ds(j * L, L)] = iv
                        cv = cv_b[pl.ds(j * L, L)]
                        _rmw_min(grid_v, iv, cv, slab_lo, slab_n)
                        return 0
                    lax.fori_loop(0, CH // L, vloop, 0)

                    start = pl.multiple_of(base + c * CH, 8)
                    pltpu.sync_copy(iv_b, idx_hbm.at[pl.ds(start, CH)])
                    pltpu.sync_copy(ones_v, counts_sp.at[iv_b], add=True)
            else:
                def process(c, k):
                    cv_b, iv_b = cost_b[k], bufs_i[k]

                    def vloop(j, _):
                        iv = iv_b[pl.ds(j * L, L)]
                        cv = cv_b[pl.ds(j * L, L)]
                        _rmw_min(grid_v, iv, cv, slab_lo, slab_n)
                        return 0
                    lax.fori_loop(0, CH // L, vloop, 0)

            chunk_start(p, 0, 0)

            def pair(h, _):
                c0 = 2 * h
                chunk_wait(p, 0)
                chunk_start(p, c0 + 1, 1)
                process(c0, 0)
                chunk_wait(p, 1)

                @pl.when(c0 + 2 < n_chunks)
                def _():
                    chunk_start(p, c0 + 2, 0)
                process(c0 + 1, 1)
                return 0

            lax.fori_loop(0, n_chunks // 2, pair, 0)

            pltpu.sync_copy(grid_v.at[pl.ds(0, slab_n)],
                            grids_hbm.at[wid, pl.ds(slab_lo, slab_n)])

            if p == 0:
                plsc.subcore_barrier()
                pltpu.sync_copy(counts_sp.at[pl.ds(sid * stripe, stripe)],
                                counts_hbm.at[cid, pl.ds(sid * stripe, stripe)])

    f = pl.kernel(
        body,
        out_type=(
            jax.ShapeDtypeStruct((NW, G), jnp.float32),
            jax.ShapeDtypeStruct((NC, G), jnp.int32),
            jax.ShapeDtypeStruct((n_points,), jnp.int32),
        ),
        mesh=mesh,
        compiler_params=pltpu.CompilerParams(needs_layout_passes=False),
        scratch_types=(
            pltpu.VMEM((SMAX,), jnp.float32),
            tuple(pltpu.VMEM((CH,), jnp.float32) for _ in range(6)),
            tuple(pltpu.VMEM((CH,), jnp.int32) for _ in range(2)),
            pltpu.VMEM((CH,), jnp.int32),
            pltpu.SemaphoreType.DMA,
            pltpu.VMEM_SHARED((G,), jnp.int32),
        ),
    )
    return f(xs, ys, costs)


def _merge_body(dflt_ref, grids_ref, counts_ref, cost_ref, mask_ref):
    g = grids_ref[...]
    mn = jnp.min(g, axis=0)
    c = counts_ref[0] + counts_ref[1]
    mask_ref[...] = c - 1
    cost_ref[...] = jnp.where(c >= 1, mn, dflt_ref[0, 0])


_R = 64

_merge = pl.pallas_call(
    _merge_body,
    grid=(H // _R,),
    in_specs=[
        pl.BlockSpec(memory_space=pltpu.SMEM),
        pl.BlockSpec((NW, _R, W), lambda i: (0, i, 0)),
        pl.BlockSpec((NC, _R, W), lambda i: (0, i, 0)),
    ],
    out_specs=[
        pl.BlockSpec((_R, W), lambda i: (i, 0)),
        pl.BlockSpec((_R, W), lambda i: (i, 0)),
    ],
    out_shape=[
        jax.ShapeDtypeStruct((H, W), jnp.float32),
        jax.ShapeDtypeStruct((H, W), jnp.int32),
    ],
)


def kernel(points, costs, default_cost):
    n = costs.shape[0]
    grids, counts, _ = _sc_scatter(points[:, 0], points[:, 1], costs, n_points=n)
    cost, mask = _merge(
        default_cost.reshape(1, 1),
        grids.reshape(NW, H, W),
        counts.reshape(NC, H, W),
    )
    return cost, mask


# speedup vs baseline: 18.1430x; 2.0903x over previous
import functools

import jax
import jax.numpy as jnp
from jax import lax
from jax.experimental import pallas as pl
from jax.experimental.pallas import tpu as pltpu
from jax.experimental.pallas import tpu_sc as plsc

H = 512
W = 512
G = H * W
NC = 2
NS = 16
NW = NC * NS
L = 16

CH = 2000
SLABS = ((0, 88064), (88064, 88064), (176128, 86016))
SMAX = 88064


def _fill(ref, n, vec):
    nv = n // L
    unroll = 8 if nv % 8 == 0 else (5 if nv % 5 == 0 else 1)
    step = L * unroll
    assert n % step == 0

    def body(i, _):
        for t in range(unroll):
            ref[pl.ds(i * step + t * L, L)] = vec
        return 0

    lax.fori_loop(0, n // step, body, 0)


def _slab_mask(idx_vec, slab_lo, slab_n, first, last):
    if first:
        return idx_vec < slab_lo + slab_n
    if last:
        return idx_vec >= slab_lo
    return (idx_vec >= slab_lo) & (idx_vec < slab_lo + slab_n)


def _rmw_fast(grid_ref, idx_vec, cost_vec, slab_lo, slab_n, first, last):
    m = _slab_mask(idx_vec, slab_lo, slab_n, first, last)
    li = idx_vec if slab_lo == 0 else idx_vec - slab_lo
    cur = plsc.load_gather(grid_ref, [li], mask=m)
    val = jnp.minimum(cost_vec, cur)
    plsc.store_scatter(grid_ref, [li], val, mask=m)
    chk = plsc.load_gather(grid_ref, [li], mask=m)
    return m & (chk > val)


def _rmw_min(grid_ref, idx_vec, cost_vec, slab_lo, slab_n, first, last):
    need = _rmw_fast(grid_ref, idx_vec, cost_vec, slab_lo, slab_n, first, last)
    li = idx_vec if slab_lo == 0 else idx_vec - slab_lo

    def wbody(nd):
        plsc.store_scatter(grid_ref, [li], cost_vec, mask=nd)
        c2 = plsc.load_gather(grid_ref, [li], mask=nd)
        return nd & (c2 > cost_vec)

    lax.while_loop(lambda nd: jnp.any(nd), wbody, need)


@functools.partial(jax.jit, static_argnames=("n_points",))
def _sc_scatter(xs, ys, costs, *, n_points):
    mesh = plsc.VectorSubcoreMesh(
        core_axis_name="c", subcore_axis_name="s", num_cores=NC, num_subcores=NS
    )
    pw = n_points // NW
    n_chunks = pw // CH
    assert n_chunks % 2 == 0

    def body(xs_hbm, ys_hbm, costs_hbm, grids_hbm, counts_hbm, idx_hbm,
             grid_v, bufs_f, bufs_i, ones_v, sem, sem_o0, sem_o1, counts_sp):
        sem_o = (sem_o0, sem_o1)
        cid = lax.axis_index("c")
        sid = lax.axis_index("s")
        wid = sid * NC + cid
        base = wid * pw
        inf16 = jnp.full((L,), jnp.inf, jnp.float32)

        _fill(ones_v, CH, jnp.ones((L,), jnp.int32))
        _fill(bufs_i[0], CH, jnp.zeros((L,), jnp.int32))
        stripe = G // NS
        def zloop(k, _):
            pltpu.sync_copy(bufs_i[0].at[pl.ds(0, 1024)],
                            counts_sp.at[pl.ds(sid * stripe + k * 1024, 1024)])
            return 0
        lax.fori_loop(0, stripe // 1024, zloop, 0)
        plsc.subcore_barrier()

        def chunk_start(p, c, k):
            start = pl.multiple_of(base + c * CH, 8)
            if p == 0:
                pltpu.async_copy(xs_hbm.at[pl.ds(start, CH)], bufs_f[3 * k], sem)
                pltpu.async_copy(ys_hbm.at[pl.ds(start, CH)], bufs_f[3 * k + 1], sem)
            else:
                pltpu.async_copy(idx_hbm.at[pl.ds(start, CH)], bufs_i[k], sem)
            pltpu.async_copy(costs_hbm.at[pl.ds(start, CH)], bufs_f[3 * k + 2], sem)

        def chunk_wait(p, k):
            if p == 0:
                pltpu.make_async_copy(xs_hbm.at[pl.ds(0, CH)], bufs_f[3 * k], sem).wait()
                pltpu.make_async_copy(xs_hbm.at[pl.ds(0, CH)], bufs_f[3 * k + 1], sem).wait()
            else:
                pltpu.make_async_copy(idx_hbm.at[pl.ds(0, CH)], bufs_i[k], sem).wait()
            pltpu.make_async_copy(xs_hbm.at[pl.ds(0, CH)], bufs_f[3 * k + 2], sem).wait()

        def out_issue(c, k):
            iv_b = bufs_i[k]
            start = pl.multiple_of(base + c * CH, 8)
            pltpu.async_copy(iv_b, idx_hbm.at[pl.ds(start, CH)], sem_o[k])
            pltpu.sync_copy(ones_v, counts_sp.at[iv_b], add=True)

        def out_wait(k):
            iv_b = bufs_i[k]
            pltpu.make_async_copy(iv_b, idx_hbm.at[pl.ds(0, CH)],
                                  sem_o[k]).wait()

        KU = 25
        NG = CH // L // KU

        for p, (slab_lo, slab_n) in enumerate(SLABS):
            first, last = p == 0, p == len(SLABS) - 1
            _fill(grid_v, SMAX, inf16)

            cost_b = (bufs_f[2], bufs_f[5])

            def process(c, k, p=p, slab_lo=slab_lo, slab_n=slab_n,
                        first=first, last=last):
                cv_b, iv_b = cost_b[k], bufs_i[k]
                if p == 0:
                    xv_b, yv_b = bufs_f[3 * k], bufs_f[3 * k + 1]

                def gloop(g, _):
                    j0 = g * KU
                    acc = None
                    for t in range(KU):
                        j = j0 + t
                        if p == 0:
                            xv = xv_b[pl.ds(j * L, L)]
                            yv = yv_b[pl.ds(j * L, L)]
                            ix = jnp.clip((xv + 0.5).astype(jnp.int32), 0, W - 1)
                            iy = jnp.clip((yv + 0.5).astype(jnp.int32), 0, H - 1)
                            iv = iy * W + ix
                            iv_b[pl.ds(j * L, L)] = iv
                        else:
                            iv = iv_b[pl.ds(j * L, L)]
                        cv = cv_b[pl.ds(j * L, L)]
                        need = _rmw_fast(grid_v, iv, cv, slab_lo, slab_n,
                                         first, last)
                        acc = need if acc is None else (acc | need)

                    @pl.when(jnp.any(acc))
                    def _():
                        def sloop(j, _):
                            iv = iv_b[pl.ds(j * L, L)]
                            cv = cv_b[pl.ds(j * L, L)]
                            _rmw_min(grid_v, iv, cv, slab_lo, slab_n,
                                     first, last)
                            return 0
                        lax.fori_loop(j0, j0 + KU, sloop, 0)
                    return 0

                lax.fori_loop(0, NG, gloop, 0)

            chunk_start(p, 0, 0)

            def pair(h, _, p=p, process=process):
                c0 = 2 * h
                chunk_wait(p, 0)
                chunk_start(p, c0 + 1, 1)
                if p == 0:
                    @pl.when(h > 0)
                    def _():
                        out_wait(0)
                process(c0, 0)
                if p == 0:
                    out_issue(c0, 0)
                chunk_wait(p, 1)
                if p == 0:
                    @pl.when(h > 0)
                    def _():
                        out_wait(1)
                process(c0 + 1, 1)

                @pl.when(c0 + 2 < n_chunks)
                def _():
                    chunk_start(p, c0 + 2, 0)
                if p == 0:
                    out_issue(c0 + 1, 1)
                return 0

            lax.fori_loop(0, n_chunks // 2, pair, 0)
            if p == 0:
                out_wait(0)
                out_wait(1)

            pltpu.sync_copy(grid_v.at[pl.ds(0, slab_n)],
                            grids_hbm.at[wid, pl.ds(slab_lo, slab_n)])

            if p == 0:
                plsc.subcore_barrier()
                pltpu.sync_copy(counts_sp.at[pl.ds(sid * stripe, stripe)],
                                counts_hbm.at[cid, pl.ds(sid * stripe, stripe)])

    f = pl.kernel(
        body,
        out_type=(
            jax.ShapeDtypeStruct((NW, G), jnp.float32),
            jax.ShapeDtypeStruct((NC, G), jnp.int32),
            jax.ShapeDtypeStruct((n_points,), jnp.int32),
        ),
        mesh=mesh,
        compiler_params=pltpu.CompilerParams(needs_layout_passes=False),
        scratch_types=(
            pltpu.VMEM((SMAX,), jnp.float32),
            tuple(pltpu.VMEM((CH,), jnp.float32) for _ in range(6)),
            tuple(pltpu.VMEM((CH,), jnp.int32) for _ in range(2)),
            pltpu.VMEM((CH,), jnp.int32),
            pltpu.SemaphoreType.DMA,
            pltpu.SemaphoreType.DMA,
            pltpu.SemaphoreType.DMA,
            pltpu.VMEM_SHARED((G,), jnp.int32),
        ),
    )
    return f(xs, ys, costs)


def _merge_body(dflt_ref, grids_ref, counts_ref, cost_ref, mask_ref):
    g = grids_ref[...]
    mn = jnp.min(g, axis=0)
    c = counts_ref[0] + counts_ref[1]
    mask_ref[...] = c - 1
    cost_ref[...] = jnp.where(c >= 1, mn, dflt_ref[0, 0])


_R = 64

_merge = pl.pallas_call(
    _merge_body,
    grid=(H // _R,),
    in_specs=[
        pl.BlockSpec(memory_space=pltpu.SMEM),
        pl.BlockSpec((NW, _R, W), lambda i: (0, i, 0)),
        pl.BlockSpec((NC, _R, W), lambda i: (0, i, 0)),
    ],
    out_specs=[
        pl.BlockSpec((_R, W), lambda i: (i, 0)),
        pl.BlockSpec((_R, W), lambda i: (i, 0)),
    ],
    out_shape=[
        jax.ShapeDtypeStruct((H, W), jnp.float32),
        jax.ShapeDtypeStruct((H, W), jnp.int32),
    ],
)


def kernel(points, costs, default_cost):
    n = costs.shape[0]
    grids, counts, _ = _sc_scatter(points[:, 0], points[:, 1], costs, n_points=n)
    cost, mask = _merge(
        default_cost.reshape(1, 1),
        grids.reshape(NW, H, W),
        counts.reshape(NC, H, W),
    )
    return cost, mask
